# baseline (speedup 1.0000x reference)
import functools

import jax
import jax.numpy as jnp
from jax import lax
from jax.experimental import pallas as pl
from jax.experimental.pallas import tpu as pltpu
from jax.experimental.pallas import tpu_sc as plsc

_L = 16


def _make_sc_kernel(num_nodes, table_rows, n_lambs, n_feat1):
  info = plsc.get_sparse_core_info()
  nc, ns = info.num_cores, info.num_subcores
  nw = nc * ns
  assert num_nodes % nw == 0
  bpw = num_nodes // nw
  n_feat = n_feat1 - 1
  out_w = n_lambs * n_feat

  mesh = plsc.VectorSubcoreMesh(core_axis_name="c", subcore_axis_name="s")

  @functools.partial(
      pl.kernel,
      mesh=mesh,
      out_type=jax.ShapeDtypeStruct((num_nodes, out_w), jnp.float32),
      scratch_types=[
          pltpu.VMEM((bpw,), jnp.int32),
          pltpu.VMEM((bpw, n_lambs, n_feat1), jnp.float32),
          pltpu.VMEM((bpw, out_w), jnp.float32),
          pltpu.SemaphoreType.DMA,
      ],
      compiler_params=pltpu.CompilerParams(use_tc_tiling_on_sc=False),
  )
  def sc_kernel(mem_hbm, nodes_hbm, out_hbm, idx_v, rows_v, out_v, sem):
    wid = lax.axis_index("s") * nc + lax.axis_index("c")
    base = wid * bpw
    pltpu.sync_copy(nodes_hbm.at[pl.ds(base, bpw)], idx_v)
    pltpu.async_copy(mem_hbm.at[idx_v], rows_v, sem).wait()

    def step(i, carry):
      for lamb in range(n_lambs):
        num = rows_v[i, lamb, pl.ds(0, _L)]
        shifted = rows_v[i, lamb, pl.ds(1, _L)]
        den = shifted[_L - 1]
        out_v[i, pl.ds(lamb * n_feat, _L)] = num / den
      return carry

    lax.fori_loop(0, bpw, step, 0, unroll=4)
    pltpu.sync_copy(out_v, out_hbm.at[pl.ds(base, bpw)])

  return sc_kernel


def kernel(memory, nodes):
  num_nodes = nodes.shape[0]
  table_rows, n_lambs, n_feat1 = memory.shape
  sc = _make_sc_kernel(num_nodes, table_rows, n_lambs, n_feat1)
  return sc(memory, nodes.astype(jnp.int32))


# speedup vs baseline: 3.2444x; 3.2444x over previous
import functools

import jax
import jax.numpy as jnp
from jax import lax
from jax.experimental import pallas as pl
from jax.experimental.pallas import tpu as pltpu
from jax.experimental.pallas import tpu_sc as plsc

_L = 16
_G = 16


def _make_sc_kernel(num_nodes, n_lambs, n_feat1):
  info = plsc.get_sparse_core_info()
  nc, ns = info.num_cores, info.num_subcores
  nw = nc * ns
  assert num_nodes % (nw * _G) == 0
  bpw = num_nodes // nw
  n_groups = bpw // _G
  n_feat = n_feat1 - 1
  out_w = n_lambs * n_feat

  mesh = plsc.VectorSubcoreMesh(core_axis_name="c", subcore_axis_name="s")

  @functools.partial(
      pl.kernel,
      mesh=mesh,
      out_type=jax.ShapeDtypeStruct((num_nodes, out_w), jnp.float32),
      scratch_types=[
          pltpu.VMEM((bpw,), jnp.int32),
          pltpu.VMEM((_G, n_lambs, n_feat1), jnp.float32),
          pltpu.VMEM((bpw, out_w), jnp.float32),
          pltpu.SemaphoreType.DMA,
      ],
  )
  def sc_kernel(mem_hbm, nodes_hbm, out_hbm, idx_v, stage_v, out_v, sem):
    wid = lax.axis_index("s") * nc + lax.axis_index("c")
    base = wid * bpw
    pltpu.sync_copy(nodes_hbm.at[pl.ds(base, bpw)], idx_v)

    def group(g, carry):
      idx_vec = idx_v[pl.ds(g * _G, _G)]
      for j in range(_G):
        pltpu.async_copy(mem_hbm.at[idx_vec[j]], stage_v.at[j], sem)
      pltpu.make_async_copy(mem_hbm.at[0], stage_v, sem).wait()
      for j in range(_G):
        for lamb in range(n_lambs):
          num = stage_v[j, lamb, pl.ds(0, _L)]
          shifted = stage_v[j, lamb, pl.ds(1, _L)]
          den = shifted[_L - 1]
          out_v[g * _G + j, pl.ds(lamb * n_feat, _L)] = num / den
      return carry

    lax.fori_loop(0, n_groups, group, 0)
    pltpu.sync_copy(out_v, out_hbm.at[pl.ds(base, bpw)])

  return sc_kernel


def kernel(memory, nodes):
  num_nodes = nodes.shape[0]
  _, n_lambs, n_feat1 = memory.shape
  sc = _make_sc_kernel(num_nodes, n_lambs, n_feat1)
  return sc(memory, nodes.astype(jnp.int32))
